# baseline (speedup 1.0000x reference)
import functools

import jax
import jax.numpy as jnp
from jax.experimental import pallas as pl
from jax.experimental.pallas import tpu as pltpu

_VMEM_BYTES = 48 * 1024 * 1024


def _rup(x, m):
    return ((x + m - 1) // m) * m


def _inv_spec(shape):
    ndim = len(shape)
    index_map = lambda b, _n=ndim: (0,) * _n
    if hasattr(pl, "Buffered"):
        return pl.BlockSpec(shape, index_map, pipeline_mode=pl.Buffered(1))
    return pl.BlockSpec(shape, index_map)


def _stem_kernel(x_ref, e_ref, l_ref, we_ref, wo_ref, s_ref, o_ref):
    acc = jnp.zeros((12544, 128), jnp.float32) + s_ref[...]
    for img in range(2):
        xe = jnp.dot(l_ref[0], x_ref[img],
                     preferred_element_type=jnp.float32).astype(jnp.bfloat16)
        xo = jnp.dot(l_ref[1], x_ref[img],
                     preferred_element_type=jnp.float32).astype(jnp.bfloat16)
        pe = jnp.dot(xe, e_ref[...],
                     preferred_element_type=jnp.float32).astype(jnp.bfloat16)
        po = jnp.dot(xo, e_ref[...],
                     preferred_element_type=jnp.float32).astype(jnp.bfloat16)
        ge = pe.reshape(115, 112, 21)
        go = po.reshape(115, 112, 21)
        for s in range(4):
            win = ge[s:s + 112].reshape(12544, 21)
            acc = acc + jnp.dot(win, we_ref[s, img],
                                preferred_element_type=jnp.float32)
        for s in range(3):
            win = go[s:s + 112].reshape(12544, 21)
            acc = acc + jnp.dot(win, wo_ref[s, img],
                                preferred_element_type=jnp.float32)
    y3 = acc.astype(jnp.bfloat16).reshape(112, 112, 128)

    neg = jnp.full((1, 112, 128), -jnp.inf, jnp.bfloat16)
    y4 = y3.reshape(56, 2, 112, 128)
    ev, od = y4[:, 0], y4[:, 1]
    od_up = jnp.concatenate([neg, od[:-1]], axis=0)
    vi = jnp.maximum(jnp.maximum(ev, od), od_up)

    v4 = vi.reshape(56, 56, 2, 128)
    evj, odj = v4[:, :, 0], v4[:, :, 1]
    negc = jnp.full((56, 1, 128), -jnp.inf, jnp.bfloat16)
    odj_up = jnp.concatenate([negc, odj[:, :-1]], axis=1)
    p = jnp.maximum(jnp.maximum(evj, odj), odj_up)

    zc = jnp.zeros((56, 1, 128), jnp.bfloat16)
    o_ref[...] = jnp.concatenate([zc, p, zc], axis=1).reshape(3248, 128)


def _stem(x, stem_w, stem_shift):
    n = x.shape[0]
    xp = jnp.pad(x.astype(jnp.bfloat16),
                 ((0, 0), (0, 0), (3, 3), (3, 3)))
    xm = jnp.transpose(xp, (0, 2, 1, 3)).reshape(n, 230, 690)

    src = jax.lax.broadcasted_iota(jnp.int32, (690, 1), 0)
    c_of = src // 230
    v_of = src % 230
    dst = jax.lax.broadcasted_iota(jnp.int32, (1, 2352), 1)
    j_of = dst // 21
    b_of = (dst % 21) // 3
    c_dst = dst % 3
    e12 = jnp.where(
        (v_of == 2 * j_of + b_of) & (c_of == c_dst), 1.0, 0.0
    ).astype(jnp.bfloat16)

    m_of = jax.lax.broadcasted_iota(jnp.int32, (1, 115, 1), 1)
    u_of = jax.lax.broadcasted_iota(jnp.int32, (1, 1, 230), 2)
    par = jax.lax.broadcasted_iota(jnp.int32, (2, 1, 1), 0)
    lpar = jnp.where(u_of == 2 * m_of + par, 1.0, 0.0).astype(jnp.bfloat16)

    w4 = stem_w.reshape(7, 7, 3, 128)
    wse = jnp.stack([w4[2 * s].reshape(21, 128) for s in range(4)])
    wso = jnp.stack([w4[2 * s + 1].reshape(21, 128) for s in range(3)])

    def pack_pair(w):
        wb = jnp.concatenate([jnp.zeros_like(w[..., :64]), w[..., :64]],
                             axis=-1)
        return jnp.stack([w, wb], axis=1)

    wse_p = pack_pair(wse)
    wso_p = pack_pair(wso)
    shift_p = jnp.concatenate([stem_shift[:, :64], stem_shift[:, :64]],
                              axis=-1)

    return pl.pallas_call(
        _stem_kernel,
        out_shape=jax.ShapeDtypeStruct((n // 2, 3248, 128), jnp.bfloat16),
        grid=(n // 2,),
        in_specs=[
            pl.BlockSpec((2, 230, 690), lambda b: (b, 0, 0)),
            _inv_spec((690, 2352)),
            _inv_spec((2, 115, 230)),
            _inv_spec((4, 2, 21, 128)),
            _inv_spec((3, 2, 21, 128)),
            _inv_spec((1, 128)),
        ],
        out_specs=pl.BlockSpec((None, 3248, 128), lambda b: (b, 0, 0)),
        compiler_params=pltpu.CompilerParams(
            dimension_semantics=("parallel",),
            vmem_limit_bytes=_VMEM_BYTES),
    )(xm, e12, lpar, wse_p, wso_p, shift_p)


def _conv3x3(z_ref, src, w_ref, M, P, Wp, cin):
    z_ref[pl.ds(P, M), pl.ds(0, cin)] = src
    acc = None
    for di in range(3):
        for dj in range(3):
            off = P + (di - 1) * Wp + (dj - 1)
            win = z_ref[pl.ds(off, M), pl.ds(0, cin)]
            d = jnp.dot(win, w_ref[di * 3 + dj],
                        preferred_element_type=jnp.float32)
            acc = d if acc is None else acc + d
    return acc


def _pool2x2_block(y, H, W, C):
    H2, W2 = H // 2, W // 2
    y3 = y.reshape(H, W + 2, C)[:, 1:W + 1, :]
    y4 = y3.reshape(H2, 2, W, C)
    t = jnp.maximum(y4[:, 0], y4[:, 1])
    t2 = t.reshape(H2, W2, 2, C)
    p = jnp.maximum(t2[:, :, 0], t2[:, :, 1])
    zc = jnp.zeros((H2, 1, C), p.dtype)
    return jnp.concatenate([zc, p, zc], axis=1).reshape(H2 * (W2 + 2), C)


def _stage_kernel(*refs, H, W, plan, mode):
    Wp = W + 2
    M = H * Wp
    P = _rup(Wp + 1, 8)

    it = iter(refs)
    x_ref = next(it)
    blk_refs = []
    for has_proj, cin, cout in plan:
        w1, s1, w2, s2 = next(it), next(it), next(it), next(it)
        pr = (next(it), next(it)) if has_proj else None
        blk_refs.append((w1, s1, w2, s2, pr))
    o_ref, z1_ref, z2_ref = next(it), next(it), next(it)

    col = jax.lax.broadcasted_iota(jnp.int32, (M, 1), 0) % Wp
    interior = jnp.logical_and(col >= 1, col <= W)

    z1_ref[...] = jnp.zeros_like(z1_ref)
    z2_ref[...] = jnp.zeros_like(z2_ref)

    x = x_ref[...]
    for (has_proj, cin, cout), (w1, s1, w2, s2, pr) in zip(plan, blk_refs):
        acc = _conv3x3(z1_ref, x, w1, M, P, Wp, cin) + s1[...]
        y1 = jnp.where(interior, jnp.maximum(acc, 0.0), 0.0)
        y1 = y1.astype(jnp.bfloat16)
        if pr is not None:
            idn = jnp.dot(x, pr[0][...],
                          preferred_element_type=jnp.float32) + pr[1][...]
        else:
            idn = x.astype(jnp.float32)
        acc2 = _conv3x3(z2_ref, y1, w2, M, P, Wp, cout) + s2[...] + idn
        x = jnp.where(interior, jnp.maximum(acc2, 0.0), 0.0)
        x = x.astype(jnp.bfloat16)

    if mode == "pool":
        o_ref[...] = _pool2x2_block(x, H, W, x.shape[-1])
    else:
        o_ref[...] = jnp.sum(x.astype(jnp.float32), axis=0,
                             keepdims=True) * (1.0 / 49.0)


def _stage(xb, blocks, H, W, mode):
    n = xb.shape[0]
    Wp = W + 2
    M = H * Wp
    P = _rup(Wp + 1, 8)
    plan = tuple((blk["proj"] is not None,
                  blk["w1"].shape[1], blk["w1"].shape[2]) for blk in blocks)
    cout = plan[-1][2]
    mz = _rup(M + 2 * P, 8)

    args = [xb]
    in_specs = [pl.BlockSpec((None, M, plan[0][1]), lambda b: (b, 0, 0))]
    for blk in blocks:
        for nm in ("w1", "s1", "w2", "s2"):
            args.append(blk[nm])
            in_specs.append(_inv_spec(blk[nm].shape))
        if blk["proj"] is not None:
            for a in blk["proj"]:
                args.append(a)
                in_specs.append(_inv_spec(a.shape))

    if mode == "pool":
        m2 = (H // 2) * (W // 2 + 2)
        out_shape = jax.ShapeDtypeStruct((n, m2, cout), jnp.bfloat16)
        out_spec = pl.BlockSpec((None, m2, cout), lambda b: (b, 0, 0))
    else:
        out_shape = jax.ShapeDtypeStruct((n, 1, cout), jnp.float32)
        out_spec = pl.BlockSpec((None, 1, cout), lambda b: (b, 0, 0))

    return pl.pallas_call(
        functools.partial(_stage_kernel, H=H, W=W, plan=plan, mode=mode),
        out_shape=out_shape,
        grid=(n,),
        in_specs=in_specs,
        out_specs=out_spec,
        scratch_shapes=[pltpu.VMEM((mz, cout), jnp.bfloat16),
                        pltpu.VMEM((mz, cout), jnp.bfloat16)],
        compiler_params=pltpu.CompilerParams(
            dimension_semantics=("parallel",),
            vmem_limit_bytes=_VMEM_BYTES),
    )(*args)


def _stage3_kernel(x_ref, w1a, w1b, s1, w2, s2, pja, pjb, pjs,
                   *rest, H, W, nblk):
    Wp = W + 2
    M = H * Wp
    P = _rup(Wp + 1, 8)

    blk_refs = []
    it = iter(rest)
    for _ in range(nblk - 1):
        blk_refs.append((next(it), next(it), next(it), next(it)))
    o_ref, z1_ref, z2_ref = next(it), next(it), next(it)

    col = jax.lax.broadcasted_iota(jnp.int32, (M, 1), 0) % Wp
    interior = jnp.logical_and(col >= 1, col <= W)

    z1_ref[...] = jnp.zeros_like(z1_ref)
    z2_ref[...] = jnp.zeros_like(z2_ref)

    x = x_ref[...]
    for img, (w1x, pjx) in enumerate(((w1a, pja), (w1b, pjb))):
        acc = _conv3x3(z1_ref, x, w1x, M, P, Wp, 128) + s1[...]
        y1 = jnp.where(interior, jnp.maximum(acc, 0.0), 0.0)
        y1 = y1.astype(jnp.bfloat16)
        idn = jnp.dot(x, pjx[...],
                      preferred_element_type=jnp.float32) + pjs[...]
        acc2 = _conv3x3(z2_ref, y1, w2, M, P, Wp, 128) + s2[...] + idn
        xi = jnp.where(interior, jnp.maximum(acc2, 0.0), 0.0)
        xi = xi.astype(jnp.bfloat16)
        for bw1, bs1, bw2, bs2 in blk_refs:
            acc = _conv3x3(z1_ref, xi, bw1, M, P, Wp, 128) + bs1[...]
            y1 = jnp.where(interior, jnp.maximum(acc, 0.0), 0.0)
            y1 = y1.astype(jnp.bfloat16)
            acc2 = _conv3x3(z2_ref, y1, bw2, M, P, Wp, 128) + bs2[...] \
                + xi.astype(jnp.float32)
            xi = jnp.where(interior, jnp.maximum(acc2, 0.0), 0.0)
            xi = xi.astype(jnp.bfloat16)
        o_ref[img] = _pool2x2_block(xi, H, W, 128)


def _stage3_pair(xb, b0, blocks, H, W):
    np_ = xb.shape[0]
    Wp = W + 2
    M = H * Wp
    P = _rup(Wp + 1, 8)
    mz = _rup(M + 2 * P, 8)
    m2 = (H // 2) * (W // 2 + 2)

    w1, s1, w2, s2, pj, pjs = b0
    w1b = jnp.concatenate([jnp.zeros_like(w1[:, :64, :]), w1[:, :64, :]],
                          axis=1)
    pjb = jnp.concatenate([jnp.zeros_like(pj[:64, :]), pj[:64, :]], axis=0)

    args = [xb, w1, w1b, s1, w2, s2, pj, pjb, pjs]
    in_specs = [pl.BlockSpec((None, M, 128), lambda b: (b, 0, 0))]
    for a in args[1:]:
        in_specs.append(_inv_spec(a.shape))
    for blk in blocks:
        for a in blk:
            args.append(a)
            in_specs.append(_inv_spec(a.shape))

    return pl.pallas_call(
        functools.partial(_stage3_kernel, H=H, W=W, nblk=1 + len(blocks)),
        out_shape=jax.ShapeDtypeStruct((np_, 2, m2, 128), jnp.bfloat16),
        grid=(np_,),
        in_specs=in_specs,
        out_specs=pl.BlockSpec((None, 2, m2, 128), lambda b: (b, 0, 0, 0)),
        scratch_shapes=[pltpu.VMEM((mz, 128), jnp.bfloat16),
                        pltpu.VMEM((mz, 128), jnp.bfloat16)],
        compiler_params=pltpu.CompilerParams(
            dimension_semantics=("parallel",),
            vmem_limit_bytes=_VMEM_BYTES),
    )(*args)


def _fc_kernel(x_ref, w_ref, s_ref, o_ref):
    o_ref[...] = jnp.dot(x_ref[...], w_ref[...],
                         preferred_element_type=jnp.float32) + s_ref[...]


def _fc(feat, fc_w, fc_shift):
    n = feat.shape[0]
    return pl.pallas_call(
        _fc_kernel,
        out_shape=jax.ShapeDtypeStruct((n, fc_w.shape[1]), jnp.float32),
    )(feat.astype(jnp.bfloat16), fc_w, fc_shift)


def kernel(x, stem_w, stem_shift, conv2_b0_c1_w, conv2_b0_c1_shift, conv2_b0_c2_w, conv2_b0_c2_shift, conv2_b1_c1_w, conv2_b1_c1_shift, conv2_b1_c2_w, conv2_b1_c2_shift, conv2_b2_c1_w, conv2_b2_c1_shift, conv2_b2_c2_w, conv2_b2_c2_shift, conv3_b0_c1_w, conv3_b0_c1_shift, conv3_b0_c2_w, conv3_b0_c2_shift, conv3_b0_proj_w, conv3_b0_proj_shift, conv3_b1_c1_w, conv3_b1_c1_shift, conv3_b1_c2_w, conv3_b1_c2_shift, conv3_b2_c1_w, conv3_b2_c1_shift, conv3_b2_c2_w, conv3_b2_c2_shift, conv3_b3_c1_w, conv3_b3_c1_shift, conv3_b3_c2_w, conv3_b3_c2_shift, conv4_b0_c1_w, conv4_b0_c1_shift, conv4_b0_c2_w, conv4_b0_c2_shift, conv4_b0_proj_w, conv4_b0_proj_shift, conv4_b1_c1_w, conv4_b1_c1_shift, conv4_b1_c2_w, conv4_b1_c2_shift, conv4_b2_c1_w, conv4_b2_c1_shift, conv4_b2_c2_w, conv4_b2_c2_shift, conv4_b3_c1_w, conv4_b3_c1_shift, conv4_b3_c2_w, conv4_b3_c2_shift, conv4_b4_c1_w, conv4_b4_c1_shift, conv4_b4_c2_w, conv4_b4_c2_shift, conv4_b5_c1_w, conv4_b5_c1_shift, conv4_b5_c2_w, conv4_b5_c2_shift, conv5_b0_c1_w, conv5_b0_c1_shift, conv5_b0_c2_w, conv5_b0_c2_shift, conv5_b0_proj_w, conv5_b0_proj_shift, conv5_b1_c1_w, conv5_b1_c1_shift, conv5_b1_c2_w, conv5_b1_c2_shift, conv5_b2_c1_w, conv5_b2_c1_shift, conv5_b2_c2_w, conv5_b2_c2_shift, fc_w, fc_shift):
    def blk(w1, s1, w2, s2, proj=None):
        return {"w1": w1, "s1": s1, "w2": w2, "s2": s2, "proj": proj}

    def pack_w(w):
        w64 = w[:, :64, :64]
        z = jnp.zeros_like(w64)
        return jnp.concatenate([jnp.concatenate([w64, z], axis=2),
                                jnp.concatenate([z, w64], axis=2)], axis=1)

    def pack_s(s):
        return jnp.concatenate([s[:, :64], s[:, :64]], axis=-1)

    xb = _stem(x, stem_w, stem_shift)

    xb = _stage(xb, [
        blk(pack_w(conv2_b0_c1_w), pack_s(conv2_b0_c1_shift),
            pack_w(conv2_b0_c2_w), pack_s(conv2_b0_c2_shift)),
        blk(pack_w(conv2_b1_c1_w), pack_s(conv2_b1_c1_shift),
            pack_w(conv2_b1_c2_w), pack_s(conv2_b1_c2_shift)),
        blk(pack_w(conv2_b2_c1_w), pack_s(conv2_b2_c1_shift),
            pack_w(conv2_b2_c2_w), pack_s(conv2_b2_c2_shift)),
    ], 56, 56, "pool")

    xb = _stage3_pair(
        xb,
        (conv3_b0_c1_w, conv3_b0_c1_shift, conv3_b0_c2_w, conv3_b0_c2_shift,
         conv3_b0_proj_w, conv3_b0_proj_shift),
        [
            (conv3_b1_c1_w, conv3_b1_c1_shift, conv3_b1_c2_w, conv3_b1_c2_shift),
            (conv3_b2_c1_w, conv3_b2_c1_shift, conv3_b2_c2_w, conv3_b2_c2_shift),
            (conv3_b3_c1_w, conv3_b3_c1_shift, conv3_b3_c2_w, conv3_b3_c2_shift),
        ], 28, 28)
    xb = xb.reshape(xb.shape[0] * 2, 224, 128)

    xb = _stage(xb, [
        blk(conv4_b0_c1_w, conv4_b0_c1_shift, conv4_b0_c2_w, conv4_b0_c2_shift,
            (conv4_b0_proj_w, conv4_b0_proj_shift)),
        blk(conv4_b1_c1_w, conv4_b1_c1_shift, conv4_b1_c2_w, conv4_b1_c2_shift),
        blk(conv4_b2_c1_w, conv4_b2_c1_shift, conv4_b2_c2_w, conv4_b2_c2_shift),
        blk(conv4_b3_c1_w, conv4_b3_c1_shift, conv4_b3_c2_w, conv4_b3_c2_shift),
        blk(conv4_b4_c1_w, conv4_b4_c1_shift, conv4_b4_c2_w, conv4_b4_c2_shift),
        blk(conv4_b5_c1_w, conv4_b5_c1_shift, conv4_b5_c2_w, conv4_b5_c2_shift),
    ], 14, 14, "pool")

    feat = _stage(xb, [
        blk(conv5_b0_c1_w, conv5_b0_c1_shift, conv5_b0_c2_w, conv5_b0_c2_shift,
            (conv5_b0_proj_w, conv5_b0_proj_shift)),
        blk(conv5_b1_c1_w, conv5_b1_c1_shift, conv5_b1_c2_w, conv5_b1_c2_shift),
        blk(conv5_b2_c1_w, conv5_b2_c1_shift, conv5_b2_c2_w, conv5_b2_c2_shift),
    ], 7, 7, "feat")

    logits = _fc(feat.reshape(feat.shape[0], 512), fc_w, fc_shift)
    return logits[:, :10]


# speedup vs baseline: 1.2315x; 1.2315x over previous
import functools

import jax
import jax.numpy as jnp
from jax.experimental import pallas as pl
from jax.experimental.pallas import tpu as pltpu

_VMEM_BYTES = 48 * 1024 * 1024


def _rup(x, m):
    return ((x + m - 1) // m) * m


def _inv_spec(shape):
    ndim = len(shape)
    index_map = lambda b, _n=ndim: (0,) * _n
    if hasattr(pl, "Buffered"):
        return pl.BlockSpec(shape, index_map, pipeline_mode=pl.Buffered(1))
    return pl.BlockSpec(shape, index_map)


def _stem_kernel(x_ref, e_ref, l_ref, we_ref, wo_ref, s_ref, o_ref):
    acc = jnp.zeros((12544, 128), jnp.float32) + s_ref[...]
    for img in range(2):
        xe = jnp.dot(l_ref[0], x_ref[img],
                     preferred_element_type=jnp.float32).astype(jnp.bfloat16)
        xo = jnp.dot(l_ref[1], x_ref[img],
                     preferred_element_type=jnp.float32).astype(jnp.bfloat16)
        pe = jnp.dot(xe, e_ref[...],
                     preferred_element_type=jnp.float32).astype(jnp.bfloat16)
        po = jnp.dot(xo, e_ref[...],
                     preferred_element_type=jnp.float32).astype(jnp.bfloat16)
        ge = pe.reshape(115, 112, 21)
        go = po.reshape(115, 112, 21)
        for s in range(4):
            win = ge[s:s + 112].reshape(12544, 21)
            acc = acc + jnp.dot(win, we_ref[s, img],
                                preferred_element_type=jnp.float32)
        for s in range(3):
            win = go[s:s + 112].reshape(12544, 21)
            acc = acc + jnp.dot(win, wo_ref[s, img],
                                preferred_element_type=jnp.float32)
    y3 = acc.astype(jnp.bfloat16).reshape(112, 112, 128)

    neg = jnp.full((1, 112, 128), -jnp.inf, jnp.bfloat16)
    y4 = y3.reshape(56, 2, 112, 128)
    ev, od = y4[:, 0], y4[:, 1]
    od_up = jnp.concatenate([neg, od[:-1]], axis=0)
    vi = jnp.maximum(jnp.maximum(ev, od), od_up)

    v4 = vi.reshape(56, 56, 2, 128)
    evj, odj = v4[:, :, 0], v4[:, :, 1]
    negc = jnp.full((56, 1, 128), -jnp.inf, jnp.bfloat16)
    odj_up = jnp.concatenate([negc, odj[:, :-1]], axis=1)
    p = jnp.maximum(jnp.maximum(evj, odj), odj_up)

    zl = jnp.zeros((56, 1, 128), jnp.bfloat16)
    zr = jnp.zeros((56, 7, 128), jnp.bfloat16)
    o_ref[...] = jnp.concatenate([zl, p, zr], axis=1).reshape(3584, 128)


def _stem(x, stem_w, stem_shift):
    n = x.shape[0]
    xp = jnp.pad(x.astype(jnp.bfloat16),
                 ((0, 0), (0, 0), (3, 3), (3, 3)))
    xm = jnp.transpose(xp, (0, 2, 1, 3)).reshape(n, 230, 690)

    src = jax.lax.broadcasted_iota(jnp.int32, (690, 1), 0)
    c_of = src // 230
    v_of = src % 230
    dst = jax.lax.broadcasted_iota(jnp.int32, (1, 2352), 1)
    j_of = dst // 21
    b_of = (dst % 21) // 3
    c_dst = dst % 3
    e12 = jnp.where(
        (v_of == 2 * j_of + b_of) & (c_of == c_dst), 1.0, 0.0
    ).astype(jnp.bfloat16)

    m_of = jax.lax.broadcasted_iota(jnp.int32, (1, 115, 1), 1)
    u_of = jax.lax.broadcasted_iota(jnp.int32, (1, 1, 230), 2)
    par = jax.lax.broadcasted_iota(jnp.int32, (2, 1, 1), 0)
    lpar = jnp.where(u_of == 2 * m_of + par, 1.0, 0.0).astype(jnp.bfloat16)

    w4 = stem_w.reshape(7, 7, 3, 128)
    wse = jnp.stack([w4[2 * s].reshape(21, 128) for s in range(4)])
    wso = jnp.stack([w4[2 * s + 1].reshape(21, 128) for s in range(3)])

    def pack_pair(w):
        wb = jnp.concatenate([jnp.zeros_like(w[..., :64]), w[..., :64]],
                             axis=-1)
        return jnp.stack([w, wb], axis=1)

    wse_p = pack_pair(wse)
    wso_p = pack_pair(wso)
    shift_p = jnp.concatenate([stem_shift[:, :64], stem_shift[:, :64]],
                              axis=-1)

    return pl.pallas_call(
        _stem_kernel,
        out_shape=jax.ShapeDtypeStruct((n // 2, 3584, 128), jnp.bfloat16),
        grid=(n // 2,),
        in_specs=[
            pl.BlockSpec((2, 230, 690), lambda b: (b, 0, 0)),
            _inv_spec((690, 2352)),
            _inv_spec((2, 115, 230)),
            _inv_spec((4, 2, 21, 128)),
            _inv_spec((3, 2, 21, 128)),
            _inv_spec((1, 128)),
        ],
        out_specs=pl.BlockSpec((None, 3584, 128), lambda b: (b, 0, 0)),
        compiler_params=pltpu.CompilerParams(
            dimension_semantics=("parallel",),
            vmem_limit_bytes=_VMEM_BYTES),
    )(xm, e12, lpar, wse_p, wso_p, shift_p)


def _reord(w):
    c = w.shape[1]
    return w.reshape(3, 3, c, w.shape[2]).transpose(
        1, 0, 2, 3).reshape(3, 3 * c, w.shape[2])


def _conv3x3(z_ref, src, w_ref, M, P, Wp, cin):
    for k in range(3):
        z_ref[pl.ds(P - (k - 1) * Wp, M), pl.ds(k * cin, cin)] = src
    acc = None
    for dj in range(3):
        win = z_ref[pl.ds(P + (dj - 1), M), pl.ds(0, 3 * cin)]
        d = jnp.dot(win, w_ref[dj], preferred_element_type=jnp.float32)
        acc = d if acc is None else acc + d
    return acc


def _pool2x2_block(y, H, W, Wp, wp_out, C):
    H2, W2 = H // 2, W // 2
    y3 = y.reshape(H, Wp, C)[:, 1:W + 1, :]
    y4 = y3.reshape(H2, 2, W, C)
    t = jnp.maximum(y4[:, 0], y4[:, 1])
    t2 = t.reshape(H2, W2, 2, C)
    p = jnp.maximum(t2[:, :, 0], t2[:, :, 1])
    zl = jnp.zeros((H2, 1, C), p.dtype)
    zr = jnp.zeros((H2, wp_out - W2 - 1, C), p.dtype)
    return jnp.concatenate([zl, p, zr], axis=1).reshape(H2 * wp_out, C)


def _stage_kernel(*refs, H, W, Wp, wp_out, plan, mode):
    M = H * Wp
    P = _rup(Wp + 1, 8)

    it = iter(refs)
    x_ref = next(it)
    blk_refs = []
    for has_proj, cin, cout in plan:
        w1, s1, w2, s2 = next(it), next(it), next(it), next(it)
        pr = (next(it), next(it)) if has_proj else None
        blk_refs.append((w1, s1, w2, s2, pr))
    o_ref, z1_ref, z2_ref = next(it), next(it), next(it)

    col = jax.lax.broadcasted_iota(jnp.int32, (M, 1), 0) % Wp
    interior = jnp.logical_and(col >= 1, col <= W)

    z1_ref[...] = jnp.zeros_like(z1_ref)
    z2_ref[...] = jnp.zeros_like(z2_ref)

    x = x_ref[...]
    prev_cin = plan[0][1]
    for (has_proj, cin, cout), (w1, s1, w2, s2, pr) in zip(plan, blk_refs):
        if cin != prev_cin:
            z1_ref[...] = jnp.zeros_like(z1_ref)
            prev_cin = cin
        acc = _conv3x3(z1_ref, x, w1, M, P, Wp, cin) + s1[...]
        y1 = jnp.where(interior, jnp.maximum(acc, 0.0), 0.0)
        y1 = y1.astype(jnp.bfloat16)
        if pr is not None:
            idn = jnp.dot(x, pr[0][...],
                          preferred_element_type=jnp.float32) + pr[1][...]
        else:
            idn = x.astype(jnp.float32)
        acc2 = _conv3x3(z2_ref, y1, w2, M, P, Wp, cout) + s2[...] + idn
        x = jnp.where(interior, jnp.maximum(acc2, 0.0), 0.0)
        x = x.astype(jnp.bfloat16)

    if mode == "pool":
        o_ref[...] = _pool2x2_block(x, H, W, Wp, wp_out, x.shape[-1])
    else:
        o_ref[...] = jnp.sum(x.astype(jnp.float32), axis=0,
                             keepdims=True) * (1.0 / 49.0)


def _stage(xb, blocks, H, W, wp, wp_out, mode):
    n = xb.shape[0]
    Wp = wp
    M = H * Wp
    P = _rup(Wp + 1, 8)
    plan = tuple((blk["proj"] is not None,
                  blk["w1"].shape[1] // 3, blk["w1"].shape[2])
                 for blk in blocks)
    cout = plan[-1][2]
    mz = _rup(M + P + Wp, 8)

    args = [xb]
    in_specs = [pl.BlockSpec((None, M, plan[0][1]), lambda b: (b, 0, 0))]
    for blk in blocks:
        for nm in ("w1", "s1", "w2", "s2"):
            args.append(blk[nm])
            in_specs.append(_inv_spec(blk[nm].shape))
        if blk["proj"] is not None:
            for a in blk["proj"]:
                args.append(a)
                in_specs.append(_inv_spec(a.shape))

    if mode == "pool":
        m2 = (H // 2) * wp_out
        out_shape = jax.ShapeDtypeStruct((n, m2, cout), jnp.bfloat16)
        out_spec = pl.BlockSpec((None, m2, cout), lambda b: (b, 0, 0))
    else:
        out_shape = jax.ShapeDtypeStruct((n, 1, cout), jnp.float32)
        out_spec = pl.BlockSpec((None, 1, cout), lambda b: (b, 0, 0))

    return pl.pallas_call(
        functools.partial(_stage_kernel, H=H, W=W, Wp=Wp, wp_out=wp_out,
                          plan=plan, mode=mode),
        out_shape=out_shape,
        grid=(n,),
        in_specs=in_specs,
        out_specs=out_spec,
        scratch_shapes=[pltpu.VMEM((mz, 3 * cout), jnp.bfloat16),
                        pltpu.VMEM((mz, 3 * cout), jnp.bfloat16)],
        compiler_params=pltpu.CompilerParams(
            dimension_semantics=("parallel",),
            vmem_limit_bytes=_VMEM_BYTES),
    )(*args)


def _stage3_kernel(x_ref, w1a, w1b, s1, w2, s2, pja, pjb, pjs,
                   *rest, H, W, Wp, wp_out, nblk):
    M = H * Wp
    P = _rup(Wp + 1, 8)

    blk_refs = []
    it = iter(rest)
    for _ in range(nblk - 1):
        blk_refs.append((next(it), next(it), next(it), next(it)))
    o_ref, z1_ref, z2_ref = next(it), next(it), next(it)

    col = jax.lax.broadcasted_iota(jnp.int32, (M, 1), 0) % Wp
    interior = jnp.logical_and(col >= 1, col <= W)

    z1_ref[...] = jnp.zeros_like(z1_ref)
    z2_ref[...] = jnp.zeros_like(z2_ref)

    x = x_ref[...]
    for img, (w1x, pjx) in enumerate(((w1a, pja), (w1b, pjb))):
        acc = _conv3x3(z1_ref, x, w1x, M, P, Wp, 128) + s1[...]
        y1 = jnp.where(interior, jnp.maximum(acc, 0.0), 0.0)
        y1 = y1.astype(jnp.bfloat16)
        idn = jnp.dot(x, pjx[...],
                      preferred_element_type=jnp.float32) + pjs[...]
        acc2 = _conv3x3(z2_ref, y1, w2, M, P, Wp, 128) + s2[...] + idn
        xi = jnp.where(interior, jnp.maximum(acc2, 0.0), 0.0)
        xi = xi.astype(jnp.bfloat16)
        for bw1, bs1, bw2, bs2 in blk_refs:
            acc = _conv3x3(z1_ref, xi, bw1, M, P, Wp, 128) + bs1[...]
            y1 = jnp.where(interior, jnp.maximum(acc, 0.0), 0.0)
            y1 = y1.astype(jnp.bfloat16)
            acc2 = _conv3x3(z2_ref, y1, bw2, M, P, Wp, 128) + bs2[...] \
                + xi.astype(jnp.float32)
            xi = jnp.where(interior, jnp.maximum(acc2, 0.0), 0.0)
            xi = xi.astype(jnp.bfloat16)
        o_ref[img] = _pool2x2_block(xi, H, W, Wp, wp_out, 128)


def _stage3_pair(xb, b0, blocks, H, W, wp, wp_out):
    np_ = xb.shape[0]
    Wp = wp
    M = H * Wp
    P = _rup(Wp + 1, 8)
    mz = _rup(M + P + Wp, 8)
    m2 = (H // 2) * wp_out

    w1, s1, w2, s2, pj, pjs = b0
    w1b = jnp.concatenate([jnp.zeros_like(w1[:, :64, :]), w1[:, :64, :]],
                          axis=1)
    pjb = jnp.concatenate([jnp.zeros_like(pj[:64, :]), pj[:64, :]], axis=0)
    w1, w1b, w2 = _reord(w1), _reord(w1b), _reord(w2)
    blocks = [(_reord(a), b, _reord(c), d) for a, b, c, d in blocks]

    args = [xb, w1, w1b, s1, w2, s2, pj, pjb, pjs]
    in_specs = [pl.BlockSpec((None, M, 128), lambda b: (b, 0, 0))]
    for a in args[1:]:
        in_specs.append(_inv_spec(a.shape))
    for blk in blocks:
        for a in blk:
            args.append(a)
            in_specs.append(_inv_spec(a.shape))

    return pl.pallas_call(
        functools.partial(_stage3_kernel, H=H, W=W, Wp=Wp, wp_out=wp_out,
                          nblk=1 + len(blocks)),
        out_shape=jax.ShapeDtypeStruct((np_, 2, m2, 128), jnp.bfloat16),
        grid=(np_,),
        in_specs=in_specs,
        out_specs=pl.BlockSpec((None, 2, m2, 128), lambda b: (b, 0, 0, 0)),
        scratch_shapes=[pltpu.VMEM((mz, 384), jnp.bfloat16),
                        pltpu.VMEM((mz, 384), jnp.bfloat16)],
        compiler_params=pltpu.CompilerParams(
            dimension_semantics=("parallel",),
            vmem_limit_bytes=_VMEM_BYTES),
    )(*args)


def _fc_kernel(x_ref, w_ref, s_ref, o_ref):
    o_ref[...] = jnp.dot(x_ref[...], w_ref[...],
                         preferred_element_type=jnp.float32) + s_ref[...]


def _fc(feat, fc_w, fc_shift):
    n = feat.shape[0]
    return pl.pallas_call(
        _fc_kernel,
        out_shape=jax.ShapeDtypeStruct((n, fc_w.shape[1]), jnp.float32),
    )(feat.astype(jnp.bfloat16), fc_w, fc_shift)


def kernel(x, stem_w, stem_shift, conv2_b0_c1_w, conv2_b0_c1_shift, conv2_b0_c2_w, conv2_b0_c2_shift, conv2_b1_c1_w, conv2_b1_c1_shift, conv2_b1_c2_w, conv2_b1_c2_shift, conv2_b2_c1_w, conv2_b2_c1_shift, conv2_b2_c2_w, conv2_b2_c2_shift, conv3_b0_c1_w, conv3_b0_c1_shift, conv3_b0_c2_w, conv3_b0_c2_shift, conv3_b0_proj_w, conv3_b0_proj_shift, conv3_b1_c1_w, conv3_b1_c1_shift, conv3_b1_c2_w, conv3_b1_c2_shift, conv3_b2_c1_w, conv3_b2_c1_shift, conv3_b2_c2_w, conv3_b2_c2_shift, conv3_b3_c1_w, conv3_b3_c1_shift, conv3_b3_c2_w, conv3_b3_c2_shift, conv4_b0_c1_w, conv4_b0_c1_shift, conv4_b0_c2_w, conv4_b0_c2_shift, conv4_b0_proj_w, conv4_b0_proj_shift, conv4_b1_c1_w, conv4_b1_c1_shift, conv4_b1_c2_w, conv4_b1_c2_shift, conv4_b2_c1_w, conv4_b2_c1_shift, conv4_b2_c2_w, conv4_b2_c2_shift, conv4_b3_c1_w, conv4_b3_c1_shift, conv4_b3_c2_w, conv4_b3_c2_shift, conv4_b4_c1_w, conv4_b4_c1_shift, conv4_b4_c2_w, conv4_b4_c2_shift, conv4_b5_c1_w, conv4_b5_c1_shift, conv4_b5_c2_w, conv4_b5_c2_shift, conv5_b0_c1_w, conv5_b0_c1_shift, conv5_b0_c2_w, conv5_b0_c2_shift, conv5_b0_proj_w, conv5_b0_proj_shift, conv5_b1_c1_w, conv5_b1_c1_shift, conv5_b1_c2_w, conv5_b1_c2_shift, conv5_b2_c1_w, conv5_b2_c1_shift, conv5_b2_c2_w, conv5_b2_c2_shift, fc_w, fc_shift):
    def blk(w1, s1, w2, s2, proj=None):
        return {"w1": w1, "s1": s1, "w2": w2, "s2": s2, "proj": proj}

    def pack_w(w):
        w64 = w[:, :64, :64]
        z = jnp.zeros_like(w64)
        return jnp.concatenate([jnp.concatenate([w64, z], axis=2),
                                jnp.concatenate([z, w64], axis=2)], axis=1)

    def pack_s(s):
        return jnp.concatenate([s[:, :64], s[:, :64]], axis=-1)

    xb = _stem(x, stem_w, stem_shift)

    xb = _stage(xb, [
        blk(_reord(pack_w(conv2_b0_c1_w)), pack_s(conv2_b0_c1_shift),
            _reord(pack_w(conv2_b0_c2_w)), pack_s(conv2_b0_c2_shift)),
        blk(_reord(pack_w(conv2_b1_c1_w)), pack_s(conv2_b1_c1_shift),
            _reord(pack_w(conv2_b1_c2_w)), pack_s(conv2_b1_c2_shift)),
        blk(_reord(pack_w(conv2_b2_c1_w)), pack_s(conv2_b2_c1_shift),
            _reord(pack_w(conv2_b2_c2_w)), pack_s(conv2_b2_c2_shift)),
    ], 56, 56, 64, 32, "pool")

    xb = _stage3_pair(
        xb,
        (conv3_b0_c1_w, conv3_b0_c1_shift, conv3_b0_c2_w, conv3_b0_c2_shift,
         conv3_b0_proj_w, conv3_b0_proj_shift),
        [
            (conv3_b1_c1_w, conv3_b1_c1_shift, conv3_b1_c2_w, conv3_b1_c2_shift),
            (conv3_b2_c1_w, conv3_b2_c1_shift, conv3_b2_c2_w, conv3_b2_c2_shift),
            (conv3_b3_c1_w, conv3_b3_c1_shift, conv3_b3_c2_w, conv3_b3_c2_shift),
        ], 28, 28, 32, 16)
    xb = xb.reshape(xb.shape[0] * 2, 224, 128)

    xb = _stage(xb, [
        blk(_reord(conv4_b0_c1_w), conv4_b0_c1_shift,
            _reord(conv4_b0_c2_w), conv4_b0_c2_shift,
            (conv4_b0_proj_w, conv4_b0_proj_shift)),
        blk(_reord(conv4_b1_c1_w), conv4_b1_c1_shift,
            _reord(conv4_b1_c2_w), conv4_b1_c2_shift),
        blk(_reord(conv4_b2_c1_w), conv4_b2_c1_shift,
            _reord(conv4_b2_c2_w), conv4_b2_c2_shift),
        blk(_reord(conv4_b3_c1_w), conv4_b3_c1_shift,
            _reord(conv4_b3_c2_w), conv4_b3_c2_shift),
        blk(_reord(conv4_b4_c1_w), conv4_b4_c1_shift,
            _reord(conv4_b4_c2_w), conv4_b4_c2_shift),
        blk(_reord(conv4_b5_c1_w), conv4_b5_c1_shift,
            _reord(conv4_b5_c2_w), conv4_b5_c2_shift),
    ], 14, 14, 16, 9, "pool")

    feat = _stage(xb, [
        blk(_reord(conv5_b0_c1_w), conv5_b0_c1_shift,
            _reord(conv5_b0_c2_w), conv5_b0_c2_shift,
            (conv5_b0_proj_w, conv5_b0_proj_shift)),
        blk(_reord(conv5_b1_c1_w), conv5_b1_c1_shift,
            _reord(conv5_b1_c2_w), conv5_b1_c2_shift),
        blk(_reord(conv5_b2_c1_w), conv5_b2_c1_shift,
            _reord(conv5_b2_c2_w), conv5_b2_c2_shift),
    ], 7, 7, 9, None, "feat")

    logits = _fc(feat.reshape(feat.shape[0], 512), fc_w, fc_shift)
    return logits[:, :10]
